# baseline (speedup 1.0000x reference)
import functools

import jax
import jax.numpy as jnp
from jax import lax
from jax.experimental import pallas as pl
from jax.experimental.pallas import tpu as pltpu
from jax.experimental.pallas import tpu_sc as plsc

NUM_ITEMS = 1000000
LATENT = 64
BATCH = 16384

NC = 2
NS = 16
NW = NC * NS
B_PER_W = BATCH // NW


def _make_gather():
  mesh = plsc.VectorSubcoreMesh(
      core_axis_name="c", subcore_axis_name="s", num_cores=NC,
      num_subcores=NS)

  @functools.partial(
      pl.kernel,
      mesh=mesh,
      compiler_params=pltpu.CompilerParams(use_tc_tiling_on_sc=False),
      out_type=jax.ShapeDtypeStruct((BATCH, LATENT), jnp.float32),
      scratch_types=[
          pltpu.VMEM((B_PER_W,), jnp.int32),
          pltpu.VMEM((B_PER_W, LATENT), jnp.float32),
          pltpu.SemaphoreType.DMA,
      ],
  )
  def gather_k(idx_hbm, table_hbm, out_hbm, idx_v, rows_v, sem):
    wid = lax.axis_index("s") * NC + lax.axis_index("c")
    base = wid * B_PER_W
    pltpu.sync_copy(idx_hbm.at[pl.ds(base, B_PER_W)], idx_v)
    pltpu.async_copy(table_hbm.at[idx_v], rows_v, sem).wait()
    pltpu.sync_copy(rows_v, out_hbm.at[pl.ds(base, B_PER_W)])

  return gather_k


_gather = _make_gather()

_TC_BLOCK = 1024


def _tc_body(x_ref, w_ref, b_ref, o_ref):
  x = x_ref[...]
  w = w_ref[...]
  s = jnp.sum(x * w, axis=1, keepdims=True) + b_ref[0, 0]
  o_ref[...] = jax.nn.sigmoid(s)


def _dense_stage(rows, W, b):
  grid = (BATCH // _TC_BLOCK,)
  return pl.pallas_call(
      _tc_body,
      grid=grid,
      in_specs=[
          pl.BlockSpec((_TC_BLOCK, LATENT), lambda i: (i, 0)),
          pl.BlockSpec((1, LATENT), lambda i: (0, 0)),
          pl.BlockSpec((1, 1), lambda i: (0, 0)),
      ],
      out_specs=pl.BlockSpec((_TC_BLOCK, 1), lambda i: (i, 0)),
      out_shape=jax.ShapeDtypeStruct((BATCH, 1), jnp.float32),
  )(rows, W, b.reshape(1, 1))


@jax.jit
def kernel(item_indices, emb_table, W, b):
  idx0 = (item_indices - 1).astype(jnp.int32)
  rows = _gather(idx0, emb_table)
  out = _dense_stage(rows, W, b)
  return out.reshape(BATCH)


# speedup vs baseline: 2.4791x; 2.4791x over previous
import functools

import jax
import jax.numpy as jnp
from jax import lax
from jax.experimental import pallas as pl
from jax.experimental.pallas import tpu as pltpu
from jax.experimental.pallas import tpu_sc as plsc

NUM_ITEMS = 1000000
LATENT = 64
BATCH = 16384

NC = 2
NS = 16
NW = NC * NS
B_PER_W = BATCH // NW
SUB = 8
N_TILES = NUM_ITEMS // SUB


def _make_gather():
  mesh = plsc.VectorSubcoreMesh(
      core_axis_name="c", subcore_axis_name="s", num_cores=NC,
      num_subcores=NS)

  @functools.partial(
      pl.kernel,
      mesh=mesh,
      out_type=jax.ShapeDtypeStruct((BATCH, LATENT), jnp.float32),
      scratch_types=[
          pltpu.VMEM((B_PER_W,), jnp.int32),
          pltpu.VMEM((B_PER_W, LATENT), jnp.float32),
          pltpu.SemaphoreType.DMA,
      ],
  )
  def gather_k(idx_hbm, table_hbm, out_hbm, idx_v, ext_v, sem):
    wid = lax.axis_index("s") * NC + lax.axis_index("c")
    base = wid * B_PER_W
    pltpu.sync_copy(idx_hbm.at[pl.ds(base, B_PER_W)], idx_v)

    def issue(g, _):
      vec = idx_v[pl.ds(g * 16, 16)]
      for k in range(16):
        ij = vec[k]
        t = ij // SUB
        r = ij % SUB
        pltpu.async_copy(table_hbm.at[t, r], ext_v.at[g * 16 + k], sem)
      return _

    lax.fori_loop(0, B_PER_W // 16, issue, None)
    pltpu.make_async_copy(
        out_hbm.at[pl.ds(base, B_PER_W)], ext_v, sem).wait()
    pltpu.sync_copy(ext_v, out_hbm.at[pl.ds(base, B_PER_W)])

  return gather_k


_gather = _make_gather()

_TC_BLOCK = 1024


def _tc_body(x_ref, w_ref, b_ref, o_ref):
  x = x_ref[...]
  w = w_ref[...]
  s = jnp.sum(x * w, axis=1, keepdims=True) + b_ref[0, 0]
  o_ref[...] = jax.nn.sigmoid(s)


def _dense_stage(rows, W, b):
  grid = (BATCH // _TC_BLOCK,)
  return pl.pallas_call(
      _tc_body,
      grid=grid,
      in_specs=[
          pl.BlockSpec((_TC_BLOCK, LATENT), lambda i: (i, 0)),
          pl.BlockSpec((1, LATENT), lambda i: (0, 0)),
          pl.BlockSpec((1, 1), lambda i: (0, 0)),
      ],
      out_specs=pl.BlockSpec((_TC_BLOCK, 1), lambda i: (i, 0)),
      out_shape=jax.ShapeDtypeStruct((BATCH, 1), jnp.float32),
  )(rows, W, b.reshape(1, 1))


@jax.jit
def kernel(item_indices, emb_table, W, b):
  idx0 = (item_indices - 1).astype(jnp.int32)
  table3 = emb_table.reshape(N_TILES, SUB, LATENT)
  rows = _gather(idx0, table3)
  out = _dense_stage(rows, W, b)
  return out.reshape(BATCH)
